# Initial kernel scaffold; baseline (speedup 1.0000x reference)
import jax
import jax.numpy as jnp
from jax.experimental import pallas as pl


def kernel(d, w):
    raise NotImplementedError("write your pallas kernel here")


# speedup vs baseline: 13.2497x; 13.2497x over previous
import functools
import math

import jax
import jax.numpy as jnp
import numpy as np
from jax import lax
from jax.experimental import pallas as pl
from jax.experimental.pallas import tpu as pltpu
from jax.experimental.pallas import tpu_sc as plsc

B = 32
N = 96 * 56 * 56
K = math.ceil(0.5 * N)
NC, NS, L = 2, 16, 16
CHUNK = 14336
NCH = N // CHUNK
VPC = CHUNK // L

_SIGN = np.int32(-2147483648)


def _find16(hist_ref, base, rank, total0):
    def body(j, carry):
        total, found, bin_idx, cnt_before = carry
        v = hist_ref[pl.ds(base + j * L, L)]
        s = jnp.sum(v)
        cs = plsc.cumsum(v)
        hit = jnp.logical_and(found == 0, total + s >= rank)
        below = (total + cs) < rank
        nbelow = jnp.max(plsc.all_reduce_population_count(below))
        cb = total + jnp.sum(jnp.where(below, v, 0))
        bin_idx = jnp.where(hit, j * L + nbelow, bin_idx)
        cnt_before = jnp.where(hit, cb, cnt_before)
        found = jnp.where(hit, jnp.int32(1), found)
        return total + s, found, bin_idx, cnt_before

    init = (total0, jnp.int32(0), jnp.int32(0), jnp.int32(0))
    _, _, bin_idx, cnt_before = lax.fori_loop(0, 16, body, init)
    return bin_idx, cnt_before


def _select_sc(d_rows):
    mesh = plsc.VectorSubcoreMesh(
        core_axis_name="c", subcore_axis_name="s", num_cores=NC, num_subcores=NS
    )

    @functools.partial(
        pl.kernel,
        mesh=mesh,
        out_type=jax.ShapeDtypeStruct((B, L), jnp.float32),
        compiler_params=pltpu.CompilerParams(needs_layout_passes=False),
        scratch_types=[
            pltpu.VMEM((65536,), jnp.int32),
            pltpu.VMEM((256,), jnp.int32),
            pltpu.VMEM((CHUNK,), jnp.float32),
            pltpu.VMEM((CHUNK,), jnp.float32),
            pltpu.VMEM((L,), jnp.float32),
            pltpu.SemaphoreType.DMA,
            pltpu.SemaphoreType.DMA,
        ],
    )
    def sel(d_hbm, out_hbm, hist, hist_hi, buf0, buf1, outbuf, sem0, sem1):
        row = lax.axis_index("s") * NC + lax.axis_index("c")
        bufs = (buf0, buf1)
        sems = (sem0, sem1)
        ones = jnp.ones((L,), jnp.int32)
        zeros = jnp.zeros((L,), jnp.int32)

        def zero_hists():
            def zbody(j, _):
                hist[pl.ds(j * L, L)] = zeros
                return 0
            lax.fori_loop(0, 65536 // L, zbody, 0)
            for j in range(256 // L):
                hist_hi[pl.ds(j * L, L)] = zeros

        def monotone(x):
            xi = lax.bitcast_convert_type(x, jnp.int32)
            s = lax.shift_right_arithmetic(xi, 31)
            return lax.bitwise_xor(xi, lax.bitwise_or(s, _SIGN))

        def pass1_vec(i, buf):
            u = monotone(buf[pl.ds(i * L, L)])
            plsc.addupdate_scatter(hist_hi, [lax.shift_right_logical(u, 24)], ones)
            plsc.addupdate_scatter(hist, [lax.shift_right_logical(u, 16)], ones)
            return 0

        def pass2_vec(i, buf, b16):
            u = monotone(buf[pl.ds(i * L, L)])
            m = lax.shift_right_logical(u, 16) == b16
            hi = jnp.bitwise_and(lax.shift_right_logical(u, 8), 255)
            lo = jnp.bitwise_and(u, 65535)
            plsc.addupdate_scatter(hist_hi, [hi], ones, mask=m)
            plsc.addupdate_scatter(hist, [lo], ones, mask=m)
            return 0

        def stream_pass(vec_fn):
            handles = [None] * NCH
            handles[0] = pltpu.async_copy(d_hbm.at[row * NCH], bufs[0], sems[0])
            for c in range(NCH):
                if c + 1 < NCH:
                    handles[c + 1] = pltpu.async_copy(
                        d_hbm.at[row * NCH + c + 1],
                        bufs[(c + 1) % 2],
                        sems[(c + 1) % 2],
                    )
                handles[c].wait()
                buf = bufs[c % 2]
                lax.fori_loop(0, VPC, lambda i, _, b=buf: vec_fn(i, b), 0)

        zero_hists()
        stream_pass(pass1_vec)
        bhi, cb = _find16(hist_hi, 0, jnp.int32(K), jnp.int32(0))
        b16, cb2 = _find16(hist, bhi * 256, jnp.int32(K), cb)
        b16 = bhi * 256 + b16

        zero_hists()
        stream_pass(lambda i, b: pass2_vec(i, b, b16))
        rank2 = jnp.int32(K) - cb2
        blo_hi, cb3 = _find16(hist_hi, 0, rank2, jnp.int32(0))
        blo, _ = _find16(hist, blo_hi * 256, rank2, cb3)
        blo = blo_hi * 256 + blo

        thr_u = jnp.bitwise_or(lax.shift_left(b16, 16), blo)
        orig = jnp.where(
            thr_u < 0,
            lax.bitwise_xor(thr_u, _SIGN),
            jnp.bitwise_not(thr_u),
        )
        outbuf[...] = lax.bitcast_convert_type(
            jnp.broadcast_to(orig, (L,)), jnp.float32
        )
        pltpu.sync_copy(outbuf, out_hbm.at[row])

    return sel(d_rows)


def _apply_tc(d3, thr3, w3):
    def body(thr_ref, w_ref, d_ref, o_ref):
        t = jnp.minimum(thr_ref[...], w_ref[...])
        wv = w_ref[...]
        x = d_ref[...]
        o_ref[...] = 1.0 - jnp.where(x > t, wv, x) / wv

    rows, sub = d3.shape[0], d3.shape[1]
    half = sub // 2
    return pl.pallas_call(
        body,
        grid=(rows, 2),
        in_specs=[
            pl.BlockSpec((1, 1, 128), lambda i, j: (i, 0, 0)),
            pl.BlockSpec((1, 1, 128), lambda i, j: (0, 0, 0)),
            pl.BlockSpec((1, half, 128), lambda i, j: (i, j, 0)),
        ],
        out_specs=pl.BlockSpec((1, half, 128), lambda i, j: (i, j, 0)),
        out_shape=jax.ShapeDtypeStruct(d3.shape, jnp.float32),
    )(thr3, w3, d3)


def kernel(d, w):
    d_rows = d.reshape(B * NCH, CHUNK)
    thr = _select_sc(d_rows)
    thr3 = jnp.broadcast_to(thr[:, :1], (B, 128)).reshape(B, 1, 128)
    w3 = jnp.broadcast_to(w.reshape(1, 1, 1), (1, 1, 128))
    d3 = d.reshape(B, N // 128, 128)
    out = _apply_tc(d3, thr3, w3)
    return out.reshape(d.shape)


# speedup vs baseline: 23.1632x; 1.7482x over previous
import functools
import math

import jax
import jax.numpy as jnp
import numpy as np
from jax import lax
from jax.experimental import pallas as pl
from jax.experimental.pallas import tpu as pltpu
from jax.experimental.pallas import tpu_sc as plsc

B = 32
N = 96 * 56 * 56
K = math.ceil(0.5 * N)
NC, NS, L = 2, 16, 16
CHUNK = 14336
NCH = N // CHUNK
VPC = CHUNK // L

_SIGN = np.int32(-2147483648)


def _find_group(hist_ref, rank, total0):
    def body(g, carry):
        total, found, grp, cnt_before = carry
        acc = hist_ref[pl.ds(g * 256, L)]
        for j in range(1, 16):
            acc = acc + hist_ref[pl.ds(g * 256 + j * L, L)]
        s = jnp.sum(acc)
        hit = jnp.logical_and(found == 0, total + s >= rank)
        grp = jnp.where(hit, g, grp)
        cnt_before = jnp.where(hit, total, cnt_before)
        found = jnp.where(hit, jnp.int32(1), found)
        return total + s, found, grp, cnt_before

    init = (total0, jnp.int32(0), jnp.int32(0), jnp.int32(0))
    _, _, grp, cnt_before = lax.fori_loop(0, 256, body, init)
    return grp, cnt_before


def _find16(hist_ref, base, rank, total0):
    def body(j, carry):
        total, found, bin_idx, cnt_before = carry
        v = hist_ref[pl.ds(base + j * L, L)]
        s = jnp.sum(v)
        cs = plsc.cumsum(v)
        hit = jnp.logical_and(found == 0, total + s >= rank)
        below = (total + cs) < rank
        nbelow = jnp.max(plsc.all_reduce_population_count(below))
        cb = total + jnp.sum(jnp.where(below, v, 0))
        bin_idx = jnp.where(hit, j * L + nbelow, bin_idx)
        cnt_before = jnp.where(hit, cb, cnt_before)
        found = jnp.where(hit, jnp.int32(1), found)
        return total + s, found, bin_idx, cnt_before

    init = (total0, jnp.int32(0), jnp.int32(0), jnp.int32(0))
    _, _, bin_idx, cnt_before = lax.fori_loop(0, 16, body, init)
    return bin_idx, cnt_before


def _select_sc(d_rows):
    mesh = plsc.VectorSubcoreMesh(
        core_axis_name="c", subcore_axis_name="s", num_cores=NC, num_subcores=NS
    )

    @functools.partial(
        pl.kernel,
        mesh=mesh,
        out_type=jax.ShapeDtypeStruct((B, L), jnp.float32),
        compiler_params=pltpu.CompilerParams(needs_layout_passes=False),
        scratch_types=[
            pltpu.VMEM((65536,), jnp.int32),
            pltpu.VMEM((CHUNK,), jnp.float32),
            pltpu.VMEM((CHUNK,), jnp.float32),
            pltpu.VMEM((L,), jnp.float32),
            pltpu.SemaphoreType.DMA,
            pltpu.SemaphoreType.DMA,
        ],
    )
    def sel(d_hbm, out_hbm, hist, buf0, buf1, outbuf, sem0, sem1):
        row = lax.axis_index("s") * NC + lax.axis_index("c")
        bufs = (buf0, buf1)
        sems = (sem0, sem1)
        ones = jnp.ones((L,), jnp.int32)
        zeros = jnp.zeros((L,), jnp.int32)

        def zero_hist():
            @plsc.parallel_loop(0, 65536, L, unroll=8)
            def _(j):
                hist[pl.ds(j, L)] = zeros

        def monotone(x):
            xi = lax.bitcast_convert_type(x, jnp.int32)
            s = lax.shift_right_arithmetic(xi, 31)
            return lax.bitwise_xor(xi, lax.bitwise_or(s, _SIGN))

        def pass1_vec(i, buf):
            u = monotone(buf[pl.ds(i, L)])
            plsc.addupdate_scatter(hist, [lax.shift_right_logical(u, 16)], ones)

        def pass2_vec(i, buf, b16):
            u = monotone(buf[pl.ds(i, L)])
            m = lax.shift_right_logical(u, 16) == b16
            lo = jnp.bitwise_and(u, 65535)
            plsc.addupdate_scatter(hist, [lo], ones, mask=m)

        def stream_pass(vec_fn):
            handles = [None] * NCH
            handles[0] = pltpu.async_copy(d_hbm.at[row * NCH], bufs[0], sems[0])
            for c in range(NCH):
                if c + 1 < NCH:
                    handles[c + 1] = pltpu.async_copy(
                        d_hbm.at[row * NCH + c + 1],
                        bufs[(c + 1) % 2],
                        sems[(c + 1) % 2],
                    )
                handles[c].wait()
                buf = bufs[c % 2]

                @plsc.parallel_loop(0, CHUNK, L, unroll=8)
                def _(i, b=buf):
                    vec_fn(i, b)

        zero_hist()
        stream_pass(pass1_vec)
        bhi, cb = _find_group(hist, jnp.int32(K), jnp.int32(0))
        b16, cb2 = _find16(hist, bhi * 256, jnp.int32(K), cb)
        b16 = bhi * 256 + b16

        zero_hist()
        stream_pass(lambda i, b: pass2_vec(i, b, b16))
        rank2 = jnp.int32(K) - cb2
        blo_hi, cb3 = _find_group(hist, rank2, jnp.int32(0))
        blo, _ = _find16(hist, blo_hi * 256, rank2, cb3)
        blo = blo_hi * 256 + blo

        thr_u = jnp.bitwise_or(lax.shift_left(b16, 16), blo)
        orig = jnp.where(
            thr_u < 0,
            lax.bitwise_xor(thr_u, _SIGN),
            jnp.bitwise_not(thr_u),
        )
        outbuf[...] = lax.bitcast_convert_type(
            jnp.broadcast_to(orig, (L,)), jnp.float32
        )
        pltpu.sync_copy(outbuf, out_hbm.at[row])

    return sel(d_rows)


def _apply_tc(d3, thr3, w3):
    def body(thr_ref, w_ref, d_ref, o_ref):
        t = jnp.minimum(thr_ref[...], w_ref[...])
        wv = w_ref[...]
        x = d_ref[...]
        o_ref[...] = 1.0 - jnp.where(x > t, wv, x) / wv

    rows, sub = d3.shape[0], d3.shape[1]
    half = sub // 2
    return pl.pallas_call(
        body,
        grid=(rows, 2),
        in_specs=[
            pl.BlockSpec((1, 1, 128), lambda i, j: (i, 0, 0)),
            pl.BlockSpec((1, 1, 128), lambda i, j: (0, 0, 0)),
            pl.BlockSpec((1, half, 128), lambda i, j: (i, j, 0)),
        ],
        out_specs=pl.BlockSpec((1, half, 128), lambda i, j: (i, j, 0)),
        out_shape=jax.ShapeDtypeStruct(d3.shape, jnp.float32),
    )(thr3, w3, d3)


def kernel(d, w):
    d_rows = d.reshape(B * NCH, CHUNK)
    thr = _select_sc(d_rows)
    thr3 = jnp.broadcast_to(thr[:, :1], (B, 128)).reshape(B, 1, 128)
    w3 = jnp.broadcast_to(w.reshape(1, 1, 1), (1, 1, 128))
    d3 = d.reshape(B, N // 128, 128)
    out = _apply_tc(d3, thr3, w3)
    return out.reshape(d.shape)


# speedup vs baseline: 28.5578x; 1.2329x over previous
import functools
import math

import jax
import jax.numpy as jnp
import numpy as np
from jax import lax
from jax.experimental import pallas as pl
from jax.experimental.pallas import tpu as pltpu
from jax.experimental.pallas import tpu_sc as plsc

B = 32
N = 96 * 56 * 56
K = math.ceil(0.5 * N)
NC, NS, L = 2, 16, 16
CHUNK = 14336
NCH = N // CHUNK
VPC = CHUNK // L

_SIGN = np.int32(-2147483648)


def _find_group(hist_ref, rank, total0):
    def body(g, carry):
        total, found, grp, cnt_before = carry
        acc = hist_ref[pl.ds(g * 256, L)]
        for j in range(1, 16):
            acc = acc + hist_ref[pl.ds(g * 256 + j * L, L)]
        s = jnp.sum(acc)
        hit = jnp.logical_and(found == 0, total + s >= rank)
        grp = jnp.where(hit, g, grp)
        cnt_before = jnp.where(hit, total, cnt_before)
        found = jnp.where(hit, jnp.int32(1), found)
        return total + s, found, grp, cnt_before

    init = (total0, jnp.int32(0), jnp.int32(0), jnp.int32(0))
    _, _, grp, cnt_before = lax.fori_loop(0, 256, body, init)
    return grp, cnt_before


def _find16(hist_ref, base, rank, total0):
    def body(j, carry):
        total, found, bin_idx, cnt_before = carry
        v = hist_ref[pl.ds(base + j * L, L)]
        s = jnp.sum(v)
        cs = plsc.cumsum(v)
        hit = jnp.logical_and(found == 0, total + s >= rank)
        below = (total + cs) < rank
        nbelow = jnp.max(plsc.all_reduce_population_count(below))
        cb = total + jnp.sum(jnp.where(below, v, 0))
        bin_idx = jnp.where(hit, j * L + nbelow, bin_idx)
        cnt_before = jnp.where(hit, cb, cnt_before)
        found = jnp.where(hit, jnp.int32(1), found)
        return total + s, found, bin_idx, cnt_before

    init = (total0, jnp.int32(0), jnp.int32(0), jnp.int32(0))
    _, _, bin_idx, cnt_before = lax.fori_loop(0, 16, body, init)
    return bin_idx, cnt_before


def _select_sc(d_rows):
    mesh = plsc.VectorSubcoreMesh(
        core_axis_name="c", subcore_axis_name="s", num_cores=NC, num_subcores=NS
    )

    @functools.partial(
        pl.kernel,
        mesh=mesh,
        out_type=jax.ShapeDtypeStruct((B, L), jnp.float32),
        compiler_params=pltpu.CompilerParams(needs_layout_passes=False),
        scratch_types=[
            pltpu.VMEM((65536,), jnp.int32),
            pltpu.VMEM((CHUNK,), jnp.float32),
            pltpu.VMEM((CHUNK,), jnp.float32),
            pltpu.VMEM((L,), jnp.float32),
            pltpu.SemaphoreType.DMA,
            pltpu.SemaphoreType.DMA,
        ],
    )
    def sel(d_hbm, out_hbm, hist, buf0, buf1, outbuf, sem0, sem1):
        row = lax.axis_index("s") * NC + lax.axis_index("c")
        bufs = (buf0, buf1)
        sems = (sem0, sem1)
        ones = jnp.ones((L,), jnp.int32)
        zeros = jnp.zeros((L,), jnp.int32)

        def zero_hist():
            @plsc.parallel_loop(0, 65536, L, unroll=8)
            def _(j):
                hist[pl.ds(j, L)] = zeros

        def monotone(x):
            xi = lax.bitcast_convert_type(x, jnp.int32)
            s = lax.shift_right_arithmetic(xi, 31)
            return lax.bitwise_xor(xi, lax.bitwise_or(s, _SIGN))

        def pass1_vec(i, buf):
            u = monotone(buf[pl.ds(i, L)])
            plsc.addupdate_scatter(hist, [lax.shift_right_logical(u, 16)], ones)

        def pass2_vec(i, buf, b16):
            u = monotone(buf[pl.ds(i, L)])
            m = lax.shift_right_logical(u, 16) == b16
            lo = jnp.bitwise_and(u, 65535)
            plsc.addupdate_scatter(hist, [lo], ones, mask=m)

        def stream_pass(vec_fn):
            handles = [None] * NCH
            handles[0] = pltpu.async_copy(d_hbm.at[row * NCH], bufs[0], sems[0])
            for c in range(NCH):
                if c + 1 < NCH:
                    handles[c + 1] = pltpu.async_copy(
                        d_hbm.at[row * NCH + c + 1],
                        bufs[(c + 1) % 2],
                        sems[(c + 1) % 2],
                    )
                handles[c].wait()
                buf = bufs[c % 2]

                @plsc.parallel_loop(0, CHUNK, L, unroll=8)
                def _(i, b=buf):
                    vec_fn(i, b)

        zero_hist()
        stream_pass(pass1_vec)
        bhi, cb = _find_group(hist, jnp.int32(K), jnp.int32(0))
        b16, cb2 = _find16(hist, bhi * 256, jnp.int32(K), cb)
        b16 = bhi * 256 + b16

        zero_hist()
        stream_pass(lambda i, b: pass2_vec(i, b, b16))
        rank2 = jnp.int32(K) - cb2
        blo_hi, cb3 = _find_group(hist, rank2, jnp.int32(0))
        blo, _ = _find16(hist, blo_hi * 256, rank2, cb3)
        blo = blo_hi * 256 + blo

        thr_u = jnp.bitwise_or(lax.shift_left(b16, 16), blo)
        orig = jnp.where(
            thr_u < 0,
            lax.bitwise_xor(thr_u, _SIGN),
            jnp.bitwise_not(thr_u),
        )
        outbuf[...] = lax.bitcast_convert_type(
            jnp.broadcast_to(orig, (L,)), jnp.float32
        )
        pltpu.sync_copy(outbuf, out_hbm.at[row])

    return sel(d_rows)


def _apply_tc(d4, thr1, w):
    def body(thr_ref, w_ref, d_ref, o_ref):
        i = pl.program_id(0)
        wv = w_ref[0]
        t = jnp.minimum(thr_ref[i], wv)
        x = d_ref[...]
        o_ref[...] = 1.0 - jnp.where(x > t, wv, x) / wv

    rows, ch = d4.shape[0], d4.shape[1]
    half = ch // 2
    return pl.pallas_call(
        body,
        grid=(rows, 2),
        in_specs=[
            pl.BlockSpec(memory_space=pltpu.SMEM),
            pl.BlockSpec(memory_space=pltpu.SMEM),
            pl.BlockSpec((1, half, 56, 56), lambda i, j: (i, j, 0, 0)),
        ],
        out_specs=pl.BlockSpec((1, half, 56, 56), lambda i, j: (i, j, 0, 0)),
        out_shape=jax.ShapeDtypeStruct(d4.shape, jnp.float32),
    )(thr1, w, d4)


def kernel(d, w):
    d_rows = d.reshape(B * NCH, CHUNK)
    thr = _select_sc(d_rows)
    return _apply_tc(d, thr[:, 0], w)


# speedup vs baseline: 30.6692x; 1.0739x over previous
import functools
import math

import jax
import jax.numpy as jnp
import numpy as np
from jax import lax
from jax.experimental import pallas as pl
from jax.experimental.pallas import tpu as pltpu
from jax.experimental.pallas import tpu_sc as plsc

B = 32
N = 96 * 56 * 56
K = math.ceil(0.5 * N)
NC, NS, L = 2, 16, 16
CHUNK = 14336
NCH = N // CHUNK
VPC = CHUNK // L

_SIGN = np.int32(-2147483648)


def _find_group(hist_ref, rank, total0):
    def body(g, carry):
        total, found, grp, cnt_before = carry
        acc = hist_ref[pl.ds(g * 256, L)]
        for j in range(1, 16):
            acc = acc + hist_ref[pl.ds(g * 256 + j * L, L)]
        s = jnp.sum(acc)
        hit = jnp.logical_and(found == 0, total + s >= rank)
        grp = jnp.where(hit, g, grp)
        cnt_before = jnp.where(hit, total, cnt_before)
        found = jnp.where(hit, jnp.int32(1), found)
        return total + s, found, grp, cnt_before

    init = (total0, jnp.int32(0), jnp.int32(0), jnp.int32(0))
    _, _, grp, cnt_before = lax.fori_loop(0, 256, body, init)
    return grp, cnt_before


def _find16(hist_ref, base, rank, total0):
    def body(j, carry):
        total, found, bin_idx, cnt_before = carry
        v = hist_ref[pl.ds(base + j * L, L)]
        s = jnp.sum(v)
        cs = plsc.cumsum(v)
        hit = jnp.logical_and(found == 0, total + s >= rank)
        below = (total + cs) < rank
        nbelow = jnp.max(plsc.all_reduce_population_count(below))
        cb = total + jnp.sum(jnp.where(below, v, 0))
        bin_idx = jnp.where(hit, j * L + nbelow, bin_idx)
        cnt_before = jnp.where(hit, cb, cnt_before)
        found = jnp.where(hit, jnp.int32(1), found)
        return total + s, found, bin_idx, cnt_before

    init = (total0, jnp.int32(0), jnp.int32(0), jnp.int32(0))
    _, _, bin_idx, cnt_before = lax.fori_loop(0, 16, body, init)
    return bin_idx, cnt_before


def _select_sc(d_rows):
    mesh = plsc.VectorSubcoreMesh(
        core_axis_name="c", subcore_axis_name="s", num_cores=NC, num_subcores=NS
    )

    @functools.partial(
        pl.kernel,
        mesh=mesh,
        out_type=jax.ShapeDtypeStruct((B, L), jnp.float32),
        compiler_params=pltpu.CompilerParams(needs_layout_passes=False),
        scratch_types=[
            pltpu.VMEM((65536,), jnp.int32),
            pltpu.VMEM((CHUNK,), jnp.float32),
            pltpu.VMEM((CHUNK,), jnp.float32),
            pltpu.VMEM((L,), jnp.float32),
            pltpu.SemaphoreType.DMA,
            pltpu.SemaphoreType.DMA,
        ],
    )
    def sel(d_hbm, out_hbm, hist, buf0, buf1, outbuf, sem0, sem1):
        row = lax.axis_index("s") * NC + lax.axis_index("c")
        bufs = (buf0, buf1)
        sems = (sem0, sem1)
        ones = jnp.ones((L,), jnp.int32)
        zeros = jnp.zeros((L,), jnp.int32)

        def zero_hist():
            @plsc.parallel_loop(0, 65536, L, unroll=8)
            def _(j):
                hist[pl.ds(j, L)] = zeros

        def monotone(x):
            xi = lax.bitcast_convert_type(x, jnp.int32)
            s = lax.shift_right_arithmetic(xi, 31)
            return lax.bitwise_xor(xi, lax.bitwise_or(s, _SIGN))

        def pass1_vec(i, buf):
            u = monotone(buf[pl.ds(i, L)])
            plsc.addupdate_scatter(hist, [lax.shift_right_logical(u, 16)], ones)

        def pass2_vec(i, buf, b16):
            u = monotone(buf[pl.ds(i, L)])
            m = lax.shift_right_logical(u, 16) == b16
            lo = jnp.bitwise_and(u, 65535)
            plsc.addupdate_scatter(hist, [lo], ones, mask=m)

        def stream_pass(vec_fn):
            handles = [None] * NCH
            handles[0] = pltpu.async_copy(d_hbm.at[row * NCH], bufs[0], sems[0])
            for c in range(NCH):
                if c + 1 < NCH:
                    handles[c + 1] = pltpu.async_copy(
                        d_hbm.at[row * NCH + c + 1],
                        bufs[(c + 1) % 2],
                        sems[(c + 1) % 2],
                    )
                handles[c].wait()
                buf = bufs[c % 2]

                @plsc.parallel_loop(0, CHUNK, L, unroll=8)
                def _(i, b=buf):
                    vec_fn(i, b)

        zero_hist()
        stream_pass(pass1_vec)
        bhi, cb = _find_group(hist, jnp.int32(K), jnp.int32(0))
        b16, cb2 = _find16(hist, bhi * 256, jnp.int32(K), cb)
        b16 = bhi * 256 + b16

        zero_hist()
        stream_pass(lambda i, b: pass2_vec(i, b, b16))
        rank2 = jnp.int32(K) - cb2
        blo_hi, cb3 = _find_group(hist, rank2, jnp.int32(0))
        blo, _ = _find16(hist, blo_hi * 256, rank2, cb3)
        blo = blo_hi * 256 + blo

        thr_u = jnp.bitwise_or(lax.shift_left(b16, 16), blo)
        orig = jnp.where(
            thr_u < 0,
            lax.bitwise_xor(thr_u, _SIGN),
            jnp.bitwise_not(thr_u),
        )
        outbuf[...] = lax.bitcast_convert_type(
            jnp.broadcast_to(orig, (L,)), jnp.float32
        )
        pltpu.sync_copy(outbuf, out_hbm.at[row])

    return sel(d_rows)


def _apply_tc(d4, thr1, w):
    def body(thr_ref, w_ref, d_ref, o_ref):
        i = pl.program_id(0)
        wv = w_ref[0]
        t = jnp.minimum(thr_ref[i], wv)
        x = d_ref[...]
        o_ref[...] = 1.0 - jnp.where(x > t, wv, x) / wv

    rows, sub = d4.shape[0], d4.shape[1]
    nblk = sub // 8
    return pl.pallas_call(
        body,
        grid=(rows, nblk),
        in_specs=[
            pl.BlockSpec(memory_space=pltpu.SMEM),
            pl.BlockSpec(memory_space=pltpu.SMEM),
            pl.BlockSpec((1, 8, 56, 96), lambda i, j: (i, j, 0, 0)),
        ],
        out_specs=pl.BlockSpec((1, 8, 56, 96), lambda i, j: (i, j, 0, 0)),
        out_shape=jax.ShapeDtypeStruct(d4.shape, jnp.float32),
    )(thr1, w, d4)


def kernel(d, w):
    d_rows = d.reshape(B * NCH, CHUNK)
    thr = _select_sc(d_rows)
    d_perm = jnp.transpose(d, (0, 2, 3, 1))
    out_perm = _apply_tc(d_perm, thr[:, 0], w)
    return jnp.transpose(out_perm, (0, 3, 1, 2))


# speedup vs baseline: 65.3983x; 2.1324x over previous
import functools
import math

import jax
import jax.numpy as jnp
import numpy as np
from jax import lax
from jax.experimental import pallas as pl
from jax.experimental.pallas import tpu as pltpu
from jax.experimental.pallas import tpu_sc as plsc

B = 32
N = 96 * 56 * 56
K = math.ceil(0.5 * N)
NC, NS, L = 2, 16, 16
CHUNK = 14336
NCH = N // CHUNK
VPC = CHUNK // L

_SIGN = np.int32(-2147483648)


def _find_group(hist_ref, rank, total0):
    def body(g, carry):
        total, found, grp, cnt_before = carry
        acc = hist_ref[pl.ds(g * 256, L)]
        for j in range(1, 16):
            acc = acc + hist_ref[pl.ds(g * 256 + j * L, L)]
        s = jnp.sum(acc)
        hit = jnp.logical_and(found == 0, total + s >= rank)
        grp = jnp.where(hit, g, grp)
        cnt_before = jnp.where(hit, total, cnt_before)
        found = jnp.where(hit, jnp.int32(1), found)
        return total + s, found, grp, cnt_before

    init = (total0, jnp.int32(0), jnp.int32(0), jnp.int32(0))
    _, _, grp, cnt_before = lax.fori_loop(0, 256, body, init)
    return grp, cnt_before


def _find16(hist_ref, base, rank, total0):
    def body(j, carry):
        total, found, bin_idx, cnt_before = carry
        v = hist_ref[pl.ds(base + j * L, L)]
        s = jnp.sum(v)
        cs = plsc.cumsum(v)
        hit = jnp.logical_and(found == 0, total + s >= rank)
        below = (total + cs) < rank
        nbelow = jnp.max(plsc.all_reduce_population_count(below))
        cb = total + jnp.sum(jnp.where(below, v, 0))
        bin_idx = jnp.where(hit, j * L + nbelow, bin_idx)
        cnt_before = jnp.where(hit, cb, cnt_before)
        found = jnp.where(hit, jnp.int32(1), found)
        return total + s, found, bin_idx, cnt_before

    init = (total0, jnp.int32(0), jnp.int32(0), jnp.int32(0))
    _, _, bin_idx, cnt_before = lax.fori_loop(0, 16, body, init)
    return bin_idx, cnt_before


def _select_sc(d4):
    mesh = plsc.VectorSubcoreMesh(
        core_axis_name="c", subcore_axis_name="s", num_cores=NC, num_subcores=NS
    )
    P = 4
    NCH4 = 56 // P

    @functools.partial(
        pl.kernel,
        mesh=mesh,
        out_type=jax.ShapeDtypeStruct((B, L), jnp.float32),
        compiler_params=pltpu.CompilerParams(
            needs_layout_passes=False, use_tc_tiling_on_sc=True
        ),
        scratch_types=[
            pltpu.VMEM((65536,), jnp.int32),
            pltpu.VMEM((P, 56, 96), jnp.float32),
            pltpu.VMEM((P, 56, 96), jnp.float32),
            pltpu.VMEM((L,), jnp.float32),
            pltpu.SemaphoreType.DMA,
            pltpu.SemaphoreType.DMA,
        ],
    )
    def sel(d_hbm, out_hbm, hist, buf0, buf1, outbuf, sem0, sem1):
        row = lax.axis_index("s") * NC + lax.axis_index("c")
        bufs = (buf0, buf1)
        sems = (sem0, sem1)
        ones = jnp.ones((L,), jnp.int32)
        zeros = jnp.zeros((L,), jnp.int32)

        def zero_hist():
            @plsc.parallel_loop(0, 65536, L, unroll=8)
            def _(j):
                hist[pl.ds(j, L)] = zeros

        def monotone(x):
            xi = lax.bitcast_convert_type(x, jnp.int32)
            s = lax.shift_right_arithmetic(xi, 31)
            return lax.bitwise_xor(xi, lax.bitwise_or(s, _SIGN))

        def pass1_vec(x):
            u = monotone(x)
            plsc.addupdate_scatter(hist, [lax.shift_right_logical(u, 16)], ones)

        def pass2_vec(x, b16):
            u = monotone(x)
            m = lax.shift_right_logical(u, 16) == b16
            lo = jnp.bitwise_and(u, 65535)
            plsc.addupdate_scatter(hist, [lo], ones, mask=m)

        def stream_pass(vec_fn):
            def process(b):
                @plsc.parallel_loop(0, 56, 1)
                def _(q):
                    for p in range(P):
                        for v in range(6):
                            vec_fn(b[p, q, pl.ds(v * L, L)])

            def start(c, which):
                pltpu.async_copy(
                    d_hbm.at[row, pl.ds(c * P, P)], bufs[which], sems[which]
                )

            def wait(which):
                pltpu.make_async_copy(
                    d_hbm.at[row, pl.ds(0, P)], bufs[which], sems[which]
                ).wait()

            last = jnp.int32(NCH4 - 1)
            start(0, 0)
            start(1, 1)

            def body(it, carry):
                c0 = it * 2
                wait(0)
                process(bufs[0])
                start(jnp.minimum(c0 + 2, last), 0)
                wait(1)
                process(bufs[1])
                start(jnp.minimum(c0 + 3, last), 1)
                return carry

            lax.fori_loop(0, NCH4 // 2, body, 0)
            wait(0)
            wait(1)

        zero_hist()
        stream_pass(pass1_vec)
        bhi, cb = _find_group(hist, jnp.int32(K), jnp.int32(0))
        b16, cb2 = _find16(hist, bhi * 256, jnp.int32(K), cb)
        b16 = bhi * 256 + b16

        zero_hist()
        stream_pass(lambda x: pass2_vec(x, b16))
        rank2 = jnp.int32(K) - cb2
        blo_hi, cb3 = _find_group(hist, rank2, jnp.int32(0))
        blo, _ = _find16(hist, blo_hi * 256, rank2, cb3)
        blo = blo_hi * 256 + blo

        thr_u = jnp.bitwise_or(lax.shift_left(b16, 16), blo)
        orig = jnp.where(
            thr_u < 0,
            lax.bitwise_xor(thr_u, _SIGN),
            jnp.bitwise_not(thr_u),
        )
        outbuf[...] = lax.bitcast_convert_type(
            jnp.broadcast_to(orig, (L,)), jnp.float32
        )
        pltpu.sync_copy(outbuf, out_hbm.at[row])

    return sel(d4)


def _apply_tc(d4, thr1, w):
    def body(thr_ref, w_ref, d_ref, o_ref):
        i = pl.program_id(0)
        wv = w_ref[0]
        t = jnp.minimum(thr_ref[i], wv)
        x = d_ref[...]
        o_ref[...] = 1.0 - jnp.where(x > t, wv, x) / wv

    rows, sub = d4.shape[0], d4.shape[1]
    half = sub // 2
    return pl.pallas_call(
        body,
        grid=(rows, 2),
        in_specs=[
            pl.BlockSpec(memory_space=pltpu.SMEM),
            pl.BlockSpec(memory_space=pltpu.SMEM),
            pl.BlockSpec((1, half, 56, 96), lambda i, j: (i, j, 0, 0)),
        ],
        out_specs=pl.BlockSpec((1, half, 56, 96), lambda i, j: (i, j, 0, 0)),
        out_shape=jax.ShapeDtypeStruct(d4.shape, jnp.float32),
    )(thr1, w, d4)


def kernel(d, w):
    d_perm = jnp.transpose(d, (0, 2, 3, 1))
    thr = _select_sc(d_perm)
    out_perm = _apply_tc(d_perm, thr[:, 0], w)
    return jnp.transpose(out_perm, (0, 3, 1, 2))


# speedup vs baseline: 74.5570x; 1.1400x over previous
import functools
import math

import jax
import jax.numpy as jnp
import numpy as np
from jax import lax
from jax.experimental import pallas as pl
from jax.experimental.pallas import tpu as pltpu
from jax.experimental.pallas import tpu_sc as plsc

B = 32
N = 96 * 56 * 56
K = math.ceil(0.5 * N)
NC, NS, L = 2, 16, 16
CHUNK = 14336
NCH = N // CHUNK
VPC = CHUNK // L

_SIGN = np.int32(-2147483648)


def _find_group(hist_ref, rank, total0):
    def body(g, carry):
        total, found, grp, cnt_before = carry
        acc = hist_ref[pl.ds(g * 256, L)]
        for j in range(1, 16):
            acc = acc + hist_ref[pl.ds(g * 256 + j * L, L)]
        s = jnp.sum(acc)
        hit = jnp.logical_and(found == 0, total + s >= rank)
        grp = jnp.where(hit, g, grp)
        cnt_before = jnp.where(hit, total, cnt_before)
        found = jnp.where(hit, jnp.int32(1), found)
        return total + s, found, grp, cnt_before

    init = (total0, jnp.int32(0), jnp.int32(0), jnp.int32(0))
    _, _, grp, cnt_before = lax.fori_loop(0, 256, body, init)
    return grp, cnt_before


def _find16(hist_ref, base, rank, total0):
    def body(j, carry):
        total, found, bin_idx, cnt_before = carry
        v = hist_ref[pl.ds(base + j * L, L)]
        s = jnp.sum(v)
        cs = plsc.cumsum(v)
        hit = jnp.logical_and(found == 0, total + s >= rank)
        below = (total + cs) < rank
        nbelow = jnp.max(plsc.all_reduce_population_count(below))
        cb = total + jnp.sum(jnp.where(below, v, 0))
        bin_idx = jnp.where(hit, j * L + nbelow, bin_idx)
        cnt_before = jnp.where(hit, cb, cnt_before)
        found = jnp.where(hit, jnp.int32(1), found)
        return total + s, found, bin_idx, cnt_before

    init = (total0, jnp.int32(0), jnp.int32(0), jnp.int32(0))
    _, _, bin_idx, cnt_before = lax.fori_loop(0, 16, body, init)
    return bin_idx, cnt_before


def _fused_sc(d4, w16):
    mesh = plsc.VectorSubcoreMesh(
        core_axis_name="c", subcore_axis_name="s", num_cores=NC, num_subcores=NS
    )
    P = 4
    NCH4 = 56 // P

    @functools.partial(
        pl.kernel,
        mesh=mesh,
        out_type=jax.ShapeDtypeStruct((B, 56, 56, 96), jnp.float32),
        compiler_params=pltpu.CompilerParams(
            needs_layout_passes=False, use_tc_tiling_on_sc=True
        ),
        scratch_types=[
            pltpu.VMEM((65536,), jnp.int32),
            pltpu.VMEM((P, 56, 96), jnp.float32),
            pltpu.VMEM((P, 56, 96), jnp.float32),
            pltpu.VMEM((L,), jnp.float32),
            pltpu.SemaphoreType.DMA,
            pltpu.SemaphoreType.DMA,
            pltpu.SemaphoreType.DMA,
            pltpu.SemaphoreType.DMA,
        ],
    )
    def sel(d_hbm, w_hbm, out_hbm, hist, buf0, buf1, wbuf, sem0, sem1, osem0, osem1):
        row = lax.axis_index("s") * NC + lax.axis_index("c")
        bufs = (buf0, buf1)
        sems = (sem0, sem1)
        osems = (osem0, osem1)
        ones = jnp.ones((L,), jnp.int32)
        zeros = jnp.zeros((L,), jnp.int32)
        pltpu.sync_copy(w_hbm, wbuf)
        wv = wbuf[...]

        def zero_hist():
            @plsc.parallel_loop(0, 65536, L, unroll=8)
            def _(j):
                hist[pl.ds(j, L)] = zeros

        def monotone(x):
            xi = lax.bitcast_convert_type(x, jnp.int32)
            s = lax.shift_right_arithmetic(xi, 31)
            return lax.bitwise_xor(xi, lax.bitwise_or(s, _SIGN))

        def pass1_vec(x):
            u = monotone(x)
            plsc.addupdate_scatter(hist, [lax.shift_right_logical(u, 16)], ones)

        def pass2_vec(x, b16):
            u = monotone(x)
            m = lax.shift_right_logical(u, 16) == b16
            lo = jnp.bitwise_and(u, 65535)
            plsc.addupdate_scatter(hist, [lo], ones, mask=m)

        def stream_pass(vec_fn):
            def process(b):
                @plsc.parallel_loop(0, 56, 1)
                def _(q):
                    for p in range(P):
                        for v in range(6):
                            vec_fn(b[p, q, pl.ds(v * L, L)])

            def start(c, which):
                pltpu.async_copy(
                    d_hbm.at[row, pl.ds(c * P, P)], bufs[which], sems[which]
                )

            def wait(which):
                pltpu.make_async_copy(
                    d_hbm.at[row, pl.ds(0, P)], bufs[which], sems[which]
                ).wait()

            last = jnp.int32(NCH4 - 1)
            start(0, 0)
            start(1, 1)

            def body(it, carry):
                c0 = it * 2
                wait(0)
                process(bufs[0])
                start(jnp.minimum(c0 + 2, last), 0)
                wait(1)
                process(bufs[1])
                start(jnp.minimum(c0 + 3, last), 1)
                return carry

            lax.fori_loop(0, NCH4 // 2, body, 0)
            wait(0)
            wait(1)

        zero_hist()
        stream_pass(pass1_vec)
        bhi, cb = _find_group(hist, jnp.int32(K), jnp.int32(0))
        b16, cb2 = _find16(hist, bhi * 256, jnp.int32(K), cb)
        b16 = bhi * 256 + b16

        zero_hist()
        stream_pass(lambda x: pass2_vec(x, b16))
        rank2 = jnp.int32(K) - cb2
        blo_hi, cb3 = _find_group(hist, rank2, jnp.int32(0))
        blo, _ = _find16(hist, blo_hi * 256, rank2, cb3)
        blo = blo_hi * 256 + blo

        thr_u = jnp.bitwise_or(lax.shift_left(b16, 16), blo)
        orig = jnp.where(
            thr_u < 0,
            lax.bitwise_xor(thr_u, _SIGN),
            jnp.bitwise_not(thr_u),
        )
        thr_vec = lax.bitcast_convert_type(
            jnp.broadcast_to(orig, (L,)), jnp.float32
        )
        tvec = jnp.minimum(thr_vec, wv)

        def process3(b):
            @plsc.parallel_loop(0, 56, 1)
            def _(q):
                for p in range(P):
                    for v in range(6):
                        x = b[p, q, pl.ds(v * L, L)]
                        r = jnp.where(x > tvec, wv, x)
                        b[p, q, pl.ds(v * L, L)] = 1.0 - r / wv

        def startin(c, i):
            pltpu.async_copy(d_hbm.at[row, pl.ds(c * P, P)], bufs[i], sems[i])

        def waitin(i):
            pltpu.make_async_copy(
                d_hbm.at[row, pl.ds(0, P)], bufs[i], sems[i]
            ).wait()

        def startout(c, i):
            pltpu.async_copy(
                bufs[i], out_hbm.at[row, pl.ds(c * P, P)], osems[i]
            )

        def waitout(i):
            pltpu.make_async_copy(
                bufs[i], out_hbm.at[row, pl.ds(0, P)], osems[i]
            ).wait()

        last = jnp.int32(NCH4 - 1)
        startin(0, 0)
        startin(1, 1)

        def body3(it, carry):
            c0 = it * 2
            waitin(0)
            process3(bufs[0])
            startout(c0, 0)
            waitin(1)
            process3(bufs[1])
            startout(c0 + 1, 1)
            waitout(0)
            startin(jnp.minimum(c0 + 2, last), 0)
            waitout(1)
            startin(jnp.minimum(c0 + 3, last), 1)
            return carry

        lax.fori_loop(0, NCH4 // 2, body3, 0)
        waitin(0)
        waitin(1)

    return sel(d4, w16)


def kernel(d, w):
    d_perm = jnp.transpose(d, (0, 2, 3, 1))
    w16 = jnp.broadcast_to(w, (L,))
    out_perm = _fused_sc(d_perm, w16)
    return jnp.transpose(out_perm, (0, 3, 1, 2))


# speedup vs baseline: 78.1219x; 1.0478x over previous
import functools
import math

import jax
import jax.numpy as jnp
import numpy as np
from jax import lax
from jax.experimental import pallas as pl
from jax.experimental.pallas import tpu as pltpu
from jax.experimental.pallas import tpu_sc as plsc

B = 32
N = 96 * 56 * 56
K = math.ceil(0.5 * N)
NC, NS, L = 2, 16, 16

_SIGN = np.int32(-2147483648)


def _find_group(hist_ref, rank, total0):
    def body(g, carry):
        total, found, grp, cnt_before = carry
        acc = hist_ref[pl.ds(g * 256, L)]
        for j in range(1, 16):
            acc = acc + hist_ref[pl.ds(g * 256 + j * L, L)]
        s = jnp.sum(acc)
        hit = jnp.logical_and(found == 0, total + s >= rank)
        grp = jnp.where(hit, g, grp)
        cnt_before = jnp.where(hit, total, cnt_before)
        found = jnp.where(hit, jnp.int32(1), found)
        return total + s, found, grp, cnt_before

    init = (total0, jnp.int32(0), jnp.int32(0), jnp.int32(0))
    _, _, grp, cnt_before = lax.fori_loop(0, 256, body, init)
    return grp, cnt_before


def _find16(hist_ref, base, rank, total0):
    def body(j, carry):
        total, found, bin_idx, cnt_before = carry
        v = hist_ref[pl.ds(base + j * L, L)]
        s = jnp.sum(v)
        cs = plsc.cumsum(v)
        hit = jnp.logical_and(found == 0, total + s >= rank)
        below = (total + cs) < rank
        nbelow = jnp.max(plsc.all_reduce_population_count(below))
        cb = total + jnp.sum(jnp.where(below, v, 0))
        bin_idx = jnp.where(hit, j * L + nbelow, bin_idx)
        cnt_before = jnp.where(hit, cb, cnt_before)
        found = jnp.where(hit, jnp.int32(1), found)
        return total + s, found, bin_idx, cnt_before

    init = (total0, jnp.int32(0), jnp.int32(0), jnp.int32(0))
    _, _, bin_idx, cnt_before = lax.fori_loop(0, 16, body, init)
    return bin_idx, cnt_before


def _fused_sc(d4, w16):
    mesh = plsc.VectorSubcoreMesh(
        core_axis_name="c", subcore_axis_name="s", num_cores=NC, num_subcores=NS
    )
    P = 4
    NCH4 = 56 // P

    @functools.partial(
        pl.kernel,
        mesh=mesh,
        out_type=jax.ShapeDtypeStruct((B, 56, 56, 96), jnp.float32),
        compiler_params=pltpu.CompilerParams(
            needs_layout_passes=False, use_tc_tiling_on_sc=True
        ),
        scratch_types=[
            pltpu.VMEM((65536,), jnp.int32),
            pltpu.VMEM((P, 56, 96), jnp.float32),
            pltpu.VMEM((P, 56, 96), jnp.float32),
            pltpu.VMEM((L,), jnp.float32),
            pltpu.SemaphoreType.DMA,
            pltpu.SemaphoreType.DMA,
            pltpu.SemaphoreType.DMA,
            pltpu.SemaphoreType.DMA,
        ],
    )
    def sel(d_hbm, w_hbm, out_hbm, hist, buf0, buf1, wbuf, sem0, sem1, osem0, osem1):
        row = lax.axis_index("s") * NC + lax.axis_index("c")
        bufs = (buf0, buf1)
        sems = (sem0, sem1)
        osems = (osem0, osem1)
        ones = jnp.ones((L,), jnp.int32)
        zeros = jnp.zeros((L,), jnp.int32)
        pltpu.sync_copy(w_hbm, wbuf)
        wv = wbuf[...]

        def zero_hist():
            @plsc.parallel_loop(0, 65536, L, unroll=8)
            def _(j):
                hist[pl.ds(j, L)] = zeros

        def monotone(x):
            xi = lax.bitcast_convert_type(x, jnp.int32)
            s = lax.shift_right_arithmetic(xi, 31)
            return lax.bitwise_xor(xi, lax.bitwise_or(s, _SIGN))

        def pass1_vec(x):
            u = monotone(x)
            plsc.addupdate_scatter(hist, [lax.shift_right_logical(u, 16)], ones)

        def pass2_vec(x, b16):
            u = monotone(x)
            m = lax.shift_right_logical(u, 16) == b16
            lo = jnp.bitwise_and(u, 65535)
            plsc.addupdate_scatter(hist, [lo], ones, mask=m)

        def stream_pass(vec_fn):
            def process(b):
                @plsc.parallel_loop(0, 56, 1)
                def _(q):
                    for p in range(P):
                        for v in range(6):
                            vec_fn(b[p, q, pl.ds(v * L, L)])

            def start(c, which):
                pltpu.async_copy(
                    d_hbm.at[row, pl.ds(c * P, P)], bufs[which], sems[which]
                )

            def wait(which):
                pltpu.make_async_copy(
                    d_hbm.at[row, pl.ds(0, P)], bufs[which], sems[which]
                ).wait()

            last = jnp.int32(NCH4 - 1)
            start(0, 0)
            start(1, 1)
            zero_hist()

            def body(it, carry):
                c0 = it * 2
                wait(0)
                process(bufs[0])
                start(jnp.minimum(c0 + 2, last), 0)
                wait(1)
                process(bufs[1])
                start(jnp.minimum(c0 + 3, last), 1)
                return carry

            lax.fori_loop(0, NCH4 // 2, body, 0)
            wait(0)
            wait(1)

        stream_pass(pass1_vec)
        bhi, cb = _find_group(hist, jnp.int32(K), jnp.int32(0))
        b16, cb2 = _find16(hist, bhi * 256, jnp.int32(K), cb)
        b16 = bhi * 256 + b16

        stream_pass(lambda x: pass2_vec(x, b16))

        def startin(c, i):
            pltpu.async_copy(d_hbm.at[row, pl.ds(c * P, P)], bufs[i], sems[i])

        def waitin(i):
            pltpu.make_async_copy(
                d_hbm.at[row, pl.ds(0, P)], bufs[i], sems[i]
            ).wait()

        startin(0, 0)
        startin(1, 1)

        rank2 = jnp.int32(K) - cb2
        blo_hi, cb3 = _find_group(hist, rank2, jnp.int32(0))
        blo, _ = _find16(hist, blo_hi * 256, rank2, cb3)
        blo = blo_hi * 256 + blo

        thr_u = jnp.bitwise_or(lax.shift_left(b16, 16), blo)
        orig = jnp.where(
            thr_u < 0,
            lax.bitwise_xor(thr_u, _SIGN),
            jnp.bitwise_not(thr_u),
        )
        thr_vec = lax.bitcast_convert_type(
            jnp.broadcast_to(orig, (L,)), jnp.float32
        )
        tvec = jnp.minimum(thr_vec, wv)

        def process3(b):
            @plsc.parallel_loop(0, 56, 1)
            def _(q):
                for p in range(P):
                    for v in range(6):
                        x = b[p, q, pl.ds(v * L, L)]
                        r = jnp.where(x > tvec, wv, x)
                        b[p, q, pl.ds(v * L, L)] = 1.0 - r / wv

        def startout(c, i):
            pltpu.async_copy(
                bufs[i], out_hbm.at[row, pl.ds(c * P, P)], osems[i]
            )

        def waitout(i):
            pltpu.make_async_copy(
                bufs[i], out_hbm.at[row, pl.ds(0, P)], osems[i]
            ).wait()

        last = jnp.int32(NCH4 - 1)

        def body3(it, carry):
            c0 = it * 2
            waitin(0)
            process3(bufs[0])
            startout(c0, 0)
            waitin(1)
            process3(bufs[1])
            startout(c0 + 1, 1)
            waitout(0)
            startin(jnp.minimum(c0 + 2, last), 0)
            waitout(1)
            startin(jnp.minimum(c0 + 3, last), 1)
            return carry

        lax.fori_loop(0, NCH4 // 2, body3, 0)
        waitin(0)
        waitin(1)

    return sel(d4, w16)


def kernel(d, w):
    d_perm = jnp.transpose(d, (0, 2, 3, 1))
    w16 = jnp.broadcast_to(w, (L,))
    out_perm = _fused_sc(d_perm, w16)
    return jnp.transpose(out_perm, (0, 3, 1, 2))


# speedup vs baseline: 78.3568x; 1.0030x over previous
import functools
import math

import jax
import jax.numpy as jnp
import numpy as np
from jax import lax
from jax.experimental import pallas as pl
from jax.experimental.pallas import tpu as pltpu
from jax.experimental.pallas import tpu_sc as plsc

B = 32
N = 96 * 56 * 56
K = math.ceil(0.5 * N)
NC, NS, L = 2, 16, 16

_SIGN = np.int32(-2147483648)


def _find_group(hist_ref, rank, total0):
    def body(g, carry):
        total, found, grp, cnt_before = carry
        acc = hist_ref[pl.ds(g * 256, L)]
        for j in range(1, 16):
            acc = acc + hist_ref[pl.ds(g * 256 + j * L, L)]
        s = jnp.sum(acc)
        hit = jnp.logical_and(found == 0, total + s >= rank)
        grp = jnp.where(hit, g, grp)
        cnt_before = jnp.where(hit, total, cnt_before)
        found = jnp.where(hit, jnp.int32(1), found)
        return total + s, found, grp, cnt_before

    init = (total0, jnp.int32(0), jnp.int32(0), jnp.int32(0))
    _, _, grp, cnt_before = lax.fori_loop(0, 256, body, init)
    return grp, cnt_before


def _find16(hist_ref, base, rank, total0):
    def body(j, carry):
        total, found, bin_idx, cnt_before = carry
        v = hist_ref[pl.ds(base + j * L, L)]
        s = jnp.sum(v)
        cs = plsc.cumsum(v)
        hit = jnp.logical_and(found == 0, total + s >= rank)
        below = (total + cs) < rank
        nbelow = jnp.max(plsc.all_reduce_population_count(below))
        cb = total + jnp.sum(jnp.where(below, v, 0))
        bin_idx = jnp.where(hit, j * L + nbelow, bin_idx)
        cnt_before = jnp.where(hit, cb, cnt_before)
        found = jnp.where(hit, jnp.int32(1), found)
        return total + s, found, bin_idx, cnt_before

    init = (total0, jnp.int32(0), jnp.int32(0), jnp.int32(0))
    _, _, bin_idx, cnt_before = lax.fori_loop(0, 16, body, init)
    return bin_idx, cnt_before


def _fused_sc(d4, w16):
    mesh = plsc.VectorSubcoreMesh(
        core_axis_name="c", subcore_axis_name="s", num_cores=NC, num_subcores=NS
    )
    P = 4
    NCH4 = 56 // P

    @functools.partial(
        pl.kernel,
        mesh=mesh,
        out_type=jax.ShapeDtypeStruct((B, 56, 56, 96), jnp.float32),
        compiler_params=pltpu.CompilerParams(
            needs_layout_passes=False, use_tc_tiling_on_sc=True
        ),
        scratch_types=[
            pltpu.VMEM((65536,), jnp.int32),
            pltpu.VMEM((P, 56, 96), jnp.float32),
            pltpu.VMEM((P, 56, 96), jnp.float32),
            pltpu.VMEM((L,), jnp.float32),
            pltpu.SemaphoreType.DMA,
            pltpu.SemaphoreType.DMA,
            pltpu.SemaphoreType.DMA,
            pltpu.SemaphoreType.DMA,
        ],
    )
    def sel(d_hbm, w_hbm, out_hbm, hist, buf0, buf1, wbuf, sem0, sem1, osem0, osem1):
        row = lax.axis_index("s") * NC + lax.axis_index("c")
        bufs = (buf0, buf1)
        sems = (sem0, sem1)
        osems = (osem0, osem1)
        ones = jnp.ones((L,), jnp.int32)
        zeros = jnp.zeros((L,), jnp.int32)
        pltpu.sync_copy(w_hbm, wbuf)
        wv = wbuf[...]

        def zero_hist():
            @plsc.parallel_loop(0, 65536, L, unroll=8)
            def _(j):
                hist[pl.ds(j, L)] = zeros

        def monotone(x):
            xi = lax.bitcast_convert_type(x, jnp.int32)
            s = lax.shift_right_arithmetic(xi, 31)
            return lax.bitwise_xor(xi, lax.bitwise_or(s, _SIGN))

        def pass1_vec(x):
            u = monotone(x)
            plsc.addupdate_scatter(hist, [lax.shift_right_logical(u, 16)], ones)

        def pass2_vec(x, b16):
            u = monotone(x)
            m = lax.shift_right_logical(u, 16) == b16
            lo = jnp.bitwise_and(u, 65535)
            plsc.addupdate_scatter(hist, [lo], ones, mask=m)

        last = jnp.int32(NCH4 - 1)

        def startin(c, i):
            pltpu.async_copy(d_hbm.at[row, pl.ds(c * P, P)], bufs[i], sems[i])

        def waitin(i):
            pltpu.make_async_copy(
                d_hbm.at[row, pl.ds(0, P)], bufs[i], sems[i]
            ).wait()

        def prime():
            startin(0, 0)
            startin(1, 1)

        def stream_pass(vec_fn):
            def process(b):
                @plsc.parallel_loop(0, 56, 1)
                def _(q):
                    for p in range(P):
                        for v in range(6):
                            vec_fn(b[p, q, pl.ds(v * L, L)])

            zero_hist()

            def body(it, carry):
                c0 = it * 2
                waitin(0)
                process(bufs[0])
                startin(jnp.minimum(c0 + 2, last), 0)
                waitin(1)
                process(bufs[1])
                startin(jnp.minimum(c0 + 3, last), 1)
                return carry

            lax.fori_loop(0, NCH4 // 2, body, 0)
            waitin(0)
            waitin(1)

        prime()
        stream_pass(pass1_vec)
        prime()
        bhi, cb = _find_group(hist, jnp.int32(K), jnp.int32(0))
        b16, cb2 = _find16(hist, bhi * 256, jnp.int32(K), cb)
        b16 = bhi * 256 + b16

        stream_pass(lambda x: pass2_vec(x, b16))
        prime()

        rank2 = jnp.int32(K) - cb2
        blo_hi, cb3 = _find_group(hist, rank2, jnp.int32(0))
        blo, _ = _find16(hist, blo_hi * 256, rank2, cb3)
        blo = blo_hi * 256 + blo

        thr_u = jnp.bitwise_or(lax.shift_left(b16, 16), blo)
        orig = jnp.where(
            thr_u < 0,
            lax.bitwise_xor(thr_u, _SIGN),
            jnp.bitwise_not(thr_u),
        )
        thr_vec = lax.bitcast_convert_type(
            jnp.broadcast_to(orig, (L,)), jnp.float32
        )
        tvec = jnp.minimum(thr_vec, wv)

        def process3(b):
            @plsc.parallel_loop(0, 56, 1)
            def _(q):
                for p in range(P):
                    for v in range(6):
                        x = b[p, q, pl.ds(v * L, L)]
                        r = jnp.where(x > tvec, wv, x)
                        b[p, q, pl.ds(v * L, L)] = 1.0 - r / wv

        def startout(c, i):
            pltpu.async_copy(
                bufs[i], out_hbm.at[row, pl.ds(c * P, P)], osems[i]
            )

        def waitout(i):
            pltpu.make_async_copy(
                bufs[i], out_hbm.at[row, pl.ds(0, P)], osems[i]
            ).wait()

        def body3(it, carry):
            c0 = it * 2
            waitin(0)
            process3(bufs[0])
            startout(c0, 0)
            waitin(1)
            process3(bufs[1])
            startout(c0 + 1, 1)
            waitout(0)
            startin(jnp.minimum(c0 + 2, last), 0)
            waitout(1)
            startin(jnp.minimum(c0 + 3, last), 1)
            return carry

        lax.fori_loop(0, NCH4 // 2, body3, 0)
        waitin(0)
        waitin(1)

    return sel(d4, w16)


def kernel(d, w):
    d_perm = jnp.transpose(d, (0, 2, 3, 1))
    w16 = jnp.broadcast_to(w, (L,))
    out_perm = _fused_sc(d_perm, w16)
    return jnp.transpose(out_perm, (0, 3, 1, 2))
